# baseline (speedup 1.0000x reference)
import functools

import jax
import jax.numpy as jnp
from jax import lax
from jax.experimental import pallas as pl
from jax.experimental.pallas import tpu as pltpu
from jax.experimental.pallas import tpu_sc as plsc

N = 10000
E = N * 32
H = 128

NC = 2
NS = 16
NW = NC * NS
EPW = E // NW
C = 100
CHUNKS = EPW // C
NR = 3
ROWS_PER_TILE = 624
TAIL_ROWS = N - NS * ROWS_PER_TILE

_NEG_SLOPE = 0.2


def _lrelu(x):
    return jnp.where(x >= 0, x, _NEG_SLOPE * x)


def _make_sc_agg(compute_deg: bool):
    mesh = plsc.VectorSubcoreMesh(core_axis_name="c", subcore_axis_name="s")
    if compute_deg:
        out_type = [jax.ShapeDtypeStruct((NC, N, H), jnp.float32),
                    jax.ShapeDtypeStruct((NC, N), jnp.float32)]
    else:
        out_type = jax.ShapeDtypeStruct((NC, N, H), jnp.float32)
    scratch_types = (
        [pltpu.VMEM((2, C), jnp.int32) for _ in range(NR)]
        + [pltpu.VMEM((C, H), jnp.float32) for _ in range(NR)]
        + [
            pltpu.VMEM((128,), jnp.float32),
            pltpu.VMEM_SHARED((N, H), jnp.float32),
            pltpu.VMEM_SHARED((N,), jnp.float32),
        ]
        + [pltpu.SemaphoreType.DMA for _ in range(4 * NR)]
    )

    def body(h_hbm, idx_hbm, zrows_hbm, zdeg_hbm, *refs):
        if compute_deg:
            agg_out, deg_out = refs[0], refs[1]
            rest = refs[2:]
        else:
            agg_out = refs[0]
            deg_out = None
            rest = refs[1:]
        idxb = rest[:NR]
        rowsb = rest[NR:2 * NR]
        ones_v, agg_sh, deg_sh = rest[2 * NR:2 * NR + 3]
        sems = rest[2 * NR + 3:]
        gsem = sems[:NR]
        ssem = sems[NR:2 * NR]
        dsem = sems[2 * NR:3 * NR]
        isem = sems[3 * NR:]

        c = lax.axis_index("c")
        s = lax.axis_index("s")
        wid = c * NS + s

        for m in range(2):
            pltpu.async_copy(idx_hbm.at[wid, m], idxb[m], isem[m])

        pltpu.sync_copy(zrows_hbm,
                        agg_sh.at[pl.ds(s * ROWS_PER_TILE, ROWS_PER_TILE)])

        @pl.when(s == NS - 1)
        def _():
            pltpu.sync_copy(zrows_hbm.at[pl.ds(0, TAIL_ROWS)],
                            agg_sh.at[pl.ds(NS * ROWS_PER_TILE, TAIL_ROWS)])

        if compute_deg:
            @pl.when(s == 0)
            def _():
                pltpu.sync_copy(zdeg_hbm, deg_sh)
            one16 = jnp.ones((16,), jnp.float32)
            for j in range(8):
                ones_v[pl.ds(j * 16, 16)] = one16

        for m in range(2):
            pltpu.make_async_copy(idx_hbm.at[wid, m], idxb[m], isem[m]).wait()
            pltpu.async_copy(h_hbm.at[idxb[m].at[0]], rowsb[m], gsem[m])
        plsc.subcore_barrier()

        def emit_step(q, b, first, gather):
            bp = (b + NR - 1) % NR
            pltpu.make_async_copy(h_hbm.at[idxb[b].at[0]],
                                  rowsb[b], gsem[b]).wait()
            pltpu.async_copy(rowsb[b], agg_sh.at[idxb[b].at[1]],
                             ssem[b], add=True)
            if compute_deg:
                pltpu.async_copy(ones_v.at[pl.ds(0, C)],
                                 deg_sh.at[idxb[b].at[1]], dsem[b], add=True)
            if not first:
                pltpu.make_async_copy(rowsb[bp], agg_sh.at[idxb[bp].at[1]],
                                      ssem[bp]).wait()
                if compute_deg:
                    pltpu.make_async_copy(ones_v.at[pl.ds(0, C)],
                                          deg_sh.at[idxb[bp].at[1]],
                                          dsem[bp]).wait()

            def advance():
                pltpu.async_copy(idx_hbm.at[wid, q + 2], idxb[bp], isem[bp])
                pltpu.make_async_copy(idx_hbm.at[wid, q + 2],
                                      idxb[bp], isem[bp]).wait()
                pltpu.async_copy(h_hbm.at[idxb[bp].at[0]], rowsb[bp], gsem[bp])

            if gather == "dyn":
                pl.when(q + 2 < CHUNKS)(advance)
            elif gather:
                advance()

        for p in range(NR):
            emit_step(p, p, first=(p == 0), gather=True)

        def round_body(j, carry):
            q0 = j * NR
            for p in range(NR):
                emit_step(q0 + p, p, first=False, gather="dyn")
            return carry

        lax.fori_loop(1, (CHUNKS - 1) // NR, round_body, 0)

        q = CHUNKS - 1
        emit_step(q, q % NR, first=False, gather=False)
        b = q % NR
        pltpu.make_async_copy(rowsb[b], agg_sh.at[idxb[b].at[1]],
                              ssem[b]).wait()
        if compute_deg:
            pltpu.make_async_copy(ones_v.at[pl.ds(0, C)],
                                  deg_sh.at[idxb[b].at[1]], dsem[b]).wait()
        plsc.subcore_barrier()

        r0 = s * ROWS_PER_TILE
        pltpu.sync_copy(agg_sh.at[pl.ds(r0, ROWS_PER_TILE)],
                        agg_out.at[c, pl.ds(r0, ROWS_PER_TILE)])

        @pl.when(s == NS - 1)
        def _():
            rt = NS * ROWS_PER_TILE
            pltpu.sync_copy(agg_sh.at[pl.ds(rt, TAIL_ROWS)],
                            agg_out.at[c, pl.ds(rt, TAIL_ROWS)])

        if compute_deg:
            @pl.when(s == 0)
            def _():
                pltpu.sync_copy(deg_sh, deg_out.at[c])

    return functools.partial(pl.kernel, mesh=mesh, out_type=out_type,
                             scratch_types=scratch_types)(body)


_sc_agg_deg = _make_sc_agg(True)
_sc_agg = _make_sc_agg(False)


BLK = 1000


def _layer1_body(p_ref, deg_ref, we_ref, be_ref, w1_ref, b1_ref,
                 w2_ref, b2_ref, o_ref):
    deg = deg_ref[0] + deg_ref[1]
    aggf = (p_ref[0] + p_ref[1]) / jnp.maximum(deg, 1.0)
    agg = jnp.dot(aggf, we_ref[...],
                  preferred_element_type=jnp.float32) + be_ref[...]
    t = _lrelu(jnp.dot(agg, w1_ref[...],
                       preferred_element_type=jnp.float32) + b1_ref[...])
    o_ref[...] = _lrelu(jnp.dot(t, w2_ref[...],
                                preferred_element_type=jnp.float32) + b2_ref[...])


def _tc_layer1(partials, degp, we, be, w1, b1, w2, b2):
    d = we.shape[0]
    return pl.pallas_call(
        _layer1_body,
        grid=(N // BLK,),
        in_specs=[
            pl.BlockSpec((NC, BLK, d), lambda i: (0, i, 0)),
            pl.BlockSpec((NC, BLK, 1), lambda i: (0, i, 0)),
            pl.BlockSpec((d, H), lambda i: (0, 0)),
            pl.BlockSpec((1, H), lambda i: (0, 0)),
            pl.BlockSpec((H, H), lambda i: (0, 0)),
            pl.BlockSpec((1, H), lambda i: (0, 0)),
            pl.BlockSpec((H, H), lambda i: (0, 0)),
            pl.BlockSpec((1, H), lambda i: (0, 0)),
        ],
        out_specs=pl.BlockSpec((BLK, H), lambda i: (i, 0)),
        out_shape=jax.ShapeDtypeStruct((N, H), jnp.float32),
    )(partials, degp, we, be.reshape(1, H), w1, b1.reshape(1, H),
      w2, b2.reshape(1, H))


def _layer_ro_body(p_ref, deg_ref, w1_ref, b1_ref, w2_ref, b2_ref,
                   wro_ref, bro_ref, o_ref):
    agg = p_ref[0] + p_ref[1]
    deg = deg_ref[0] + deg_ref[1]
    agg = agg / jnp.maximum(deg, 1.0)
    t = _lrelu(jnp.dot(agg, w1_ref[...],
                       preferred_element_type=jnp.float32) + b1_ref[...])
    h = _lrelu(jnp.dot(t, w2_ref[...],
                       preferred_element_type=jnp.float32) + b2_ref[...])
    o_ref[...] = jnp.dot(h, wro_ref[...],
                         preferred_element_type=jnp.float32) + bro_ref[...]


def _tc_layer_ro(partials, degp, w1, b1, w2, b2, wro, bro):
    o = wro.shape[1]
    return pl.pallas_call(
        _layer_ro_body,
        grid=(N // BLK,),
        in_specs=[
            pl.BlockSpec((NC, BLK, H), lambda i: (0, i, 0)),
            pl.BlockSpec((NC, BLK, 1), lambda i: (0, i, 0)),
            pl.BlockSpec((H, H), lambda i: (0, 0)),
            pl.BlockSpec((1, H), lambda i: (0, 0)),
            pl.BlockSpec((H, H), lambda i: (0, 0)),
            pl.BlockSpec((1, H), lambda i: (0, 0)),
            pl.BlockSpec((H, o), lambda i: (0, 0)),
            pl.BlockSpec((1, o), lambda i: (0, 0)),
        ],
        out_specs=pl.BlockSpec((BLK, o), lambda i: (i, 0)),
        out_shape=jax.ShapeDtypeStruct((N, o), jnp.float32),
    )(partials, degp, w1, b1.reshape(1, H), w2, b2.reshape(1, H),
      wro, bro.reshape(1, o))


def kernel(feats, edge_index, W_emb, b_emb, W1_0, b1_0, W2_0, b2_0,
           W1_1, b1_1, W2_1, b2_1, W_ro, b_ro):
    idx = jnp.stack([edge_index[0].reshape(NW, CHUNKS, C),
                     edge_index[1].reshape(NW, CHUNKS, C)], axis=2)
    zrows = jnp.zeros((ROWS_PER_TILE, H), jnp.float32)
    zdeg = jnp.zeros((N,), jnp.float32)

    aggp, degp = _sc_agg_deg(feats, idx, zrows, zdeg)
    degp3 = degp.reshape(NC, N, 1)
    h1 = _tc_layer1(aggp, degp3, W_emb, b_emb, W1_0, b1_0, W2_0, b2_0)
    aggp2 = _sc_agg(h1, idx, zrows, zdeg)
    return _tc_layer_ro(aggp2, degp3, W1_1, b1_1, W2_1, b2_1, W_ro, b_ro)


# speedup vs baseline: 1.0532x; 1.0532x over previous
import functools

import jax
import jax.numpy as jnp
from jax import lax
from jax.experimental import pallas as pl
from jax.experimental.pallas import tpu as pltpu
from jax.experimental.pallas import tpu_sc as plsc

N = 10000
E = N * 32
H = 128

NC = 2
NS = 16
NW = NC * NS
EPW = E // NW
C = 125
CHUNKS = EPW // C
NR = 3
ROWS_PER_TILE = 624
TAIL_ROWS = N - NS * ROWS_PER_TILE

_NEG_SLOPE = 0.2


def _lrelu(x):
    return jnp.where(x >= 0, x, _NEG_SLOPE * x)


def _make_sc_agg(compute_deg: bool):
    mesh = plsc.VectorSubcoreMesh(core_axis_name="c", subcore_axis_name="s")
    if compute_deg:
        out_type = [jax.ShapeDtypeStruct((NC, N, H), jnp.float32),
                    jax.ShapeDtypeStruct((NC, N), jnp.float32)]
    else:
        out_type = jax.ShapeDtypeStruct((NC, N, H), jnp.float32)
    scratch_types = (
        [pltpu.VMEM((2, C), jnp.int32) for _ in range(NR)]
        + [pltpu.VMEM((C, H), jnp.float32) for _ in range(NR)]
        + [
            pltpu.VMEM((128,), jnp.float32),
            pltpu.VMEM_SHARED((N, H), jnp.float32),
            pltpu.VMEM_SHARED((N,), jnp.float32),
        ]
        + [pltpu.SemaphoreType.DMA for _ in range(4 * NR)]
    )

    def body(h_hbm, idx_hbm, zrows_hbm, zdeg_hbm, *refs):
        if compute_deg:
            agg_out, deg_out = refs[0], refs[1]
            rest = refs[2:]
        else:
            agg_out = refs[0]
            deg_out = None
            rest = refs[1:]
        idxb = rest[:NR]
        rowsb = rest[NR:2 * NR]
        ones_v, agg_sh, deg_sh = rest[2 * NR:2 * NR + 3]
        sems = rest[2 * NR + 3:]
        gsem = sems[:NR]
        ssem = sems[NR:2 * NR]
        dsem = sems[2 * NR:3 * NR]
        isem = sems[3 * NR:]

        c = lax.axis_index("c")
        s = lax.axis_index("s")
        wid = c * NS + s

        for m in range(2):
            pltpu.async_copy(idx_hbm.at[wid, m], idxb[m], isem[m])

        pltpu.sync_copy(zrows_hbm,
                        agg_sh.at[pl.ds(s * ROWS_PER_TILE, ROWS_PER_TILE)])

        @pl.when(s == NS - 1)
        def _():
            pltpu.sync_copy(zrows_hbm.at[pl.ds(0, TAIL_ROWS)],
                            agg_sh.at[pl.ds(NS * ROWS_PER_TILE, TAIL_ROWS)])

        if compute_deg:
            @pl.when(s == 0)
            def _():
                pltpu.sync_copy(zdeg_hbm, deg_sh)
            one16 = jnp.ones((16,), jnp.float32)
            for j in range(8):
                ones_v[pl.ds(j * 16, 16)] = one16

        for m in range(2):
            pltpu.make_async_copy(idx_hbm.at[wid, m], idxb[m], isem[m]).wait()
            pltpu.async_copy(h_hbm.at[idxb[m].at[0]], rowsb[m], gsem[m])
        plsc.subcore_barrier()

        def emit_step(q, b, first, gather):
            bp = (b + NR - 1) % NR
            pltpu.make_async_copy(h_hbm.at[idxb[b].at[0]],
                                  rowsb[b], gsem[b]).wait()
            pltpu.async_copy(rowsb[b], agg_sh.at[idxb[b].at[1]],
                             ssem[b], add=True)
            if compute_deg:
                pltpu.async_copy(ones_v.at[pl.ds(0, C)],
                                 deg_sh.at[idxb[b].at[1]], dsem[b], add=True)
            if not first:
                pltpu.make_async_copy(rowsb[bp], agg_sh.at[idxb[bp].at[1]],
                                      ssem[bp]).wait()
                if compute_deg:
                    pltpu.make_async_copy(ones_v.at[pl.ds(0, C)],
                                          deg_sh.at[idxb[bp].at[1]],
                                          dsem[bp]).wait()

            def advance():
                pltpu.async_copy(idx_hbm.at[wid, q + 2], idxb[bp], isem[bp])
                pltpu.make_async_copy(idx_hbm.at[wid, q + 2],
                                      idxb[bp], isem[bp]).wait()
                pltpu.async_copy(h_hbm.at[idxb[bp].at[0]], rowsb[bp], gsem[bp])

            if gather == "dyn":
                pl.when(q + 2 < CHUNKS)(advance)
            elif gather:
                advance()

        for p in range(NR):
            emit_step(p, p, first=(p == 0), gather=True)

        def round_body(j, carry):
            q0 = j * NR
            for p in range(NR):
                emit_step(q0 + p, p, first=False, gather="dyn")
            return carry

        lax.fori_loop(1, CHUNKS // NR, round_body, 0)

        for q in range(NR * (CHUNKS // NR), CHUNKS):
            emit_step(q, q % NR, first=False, gather=False)
        b = (CHUNKS - 1) % NR
        pltpu.make_async_copy(rowsb[b], agg_sh.at[idxb[b].at[1]],
                              ssem[b]).wait()
        if compute_deg:
            pltpu.make_async_copy(ones_v.at[pl.ds(0, C)],
                                  deg_sh.at[idxb[b].at[1]], dsem[b]).wait()
        plsc.subcore_barrier()

        r0 = s * ROWS_PER_TILE
        pltpu.sync_copy(agg_sh.at[pl.ds(r0, ROWS_PER_TILE)],
                        agg_out.at[c, pl.ds(r0, ROWS_PER_TILE)])

        @pl.when(s == NS - 1)
        def _():
            rt = NS * ROWS_PER_TILE
            pltpu.sync_copy(agg_sh.at[pl.ds(rt, TAIL_ROWS)],
                            agg_out.at[c, pl.ds(rt, TAIL_ROWS)])

        if compute_deg:
            @pl.when(s == 0)
            def _():
                pltpu.sync_copy(deg_sh, deg_out.at[c])

    return functools.partial(pl.kernel, mesh=mesh, out_type=out_type,
                             scratch_types=scratch_types)(body)


_sc_agg_deg = _make_sc_agg(True)
_sc_agg = _make_sc_agg(False)


BLK = 1000


def _layer1_body(p_ref, deg_ref, we_ref, be_ref, w1_ref, b1_ref,
                 w2_ref, b2_ref, o_ref):
    deg = deg_ref[0] + deg_ref[1]
    aggf = (p_ref[0] + p_ref[1]) / jnp.maximum(deg, 1.0)
    agg = jnp.dot(aggf, we_ref[...],
                  preferred_element_type=jnp.float32) + be_ref[...]
    t = _lrelu(jnp.dot(agg, w1_ref[...],
                       preferred_element_type=jnp.float32) + b1_ref[...])
    o_ref[...] = _lrelu(jnp.dot(t, w2_ref[...],
                                preferred_element_type=jnp.float32) + b2_ref[...])


def _tc_layer1(partials, degp, we, be, w1, b1, w2, b2):
    d = we.shape[0]
    return pl.pallas_call(
        _layer1_body,
        grid=(N // BLK,),
        in_specs=[
            pl.BlockSpec((NC, BLK, d), lambda i: (0, i, 0)),
            pl.BlockSpec((NC, BLK, 1), lambda i: (0, i, 0)),
            pl.BlockSpec((d, H), lambda i: (0, 0)),
            pl.BlockSpec((1, H), lambda i: (0, 0)),
            pl.BlockSpec((H, H), lambda i: (0, 0)),
            pl.BlockSpec((1, H), lambda i: (0, 0)),
            pl.BlockSpec((H, H), lambda i: (0, 0)),
            pl.BlockSpec((1, H), lambda i: (0, 0)),
        ],
        out_specs=pl.BlockSpec((BLK, H), lambda i: (i, 0)),
        out_shape=jax.ShapeDtypeStruct((N, H), jnp.float32),
    )(partials, degp, we, be.reshape(1, H), w1, b1.reshape(1, H),
      w2, b2.reshape(1, H))


def _layer_ro_body(p_ref, deg_ref, w1_ref, b1_ref, w2_ref, b2_ref,
                   wro_ref, bro_ref, o_ref):
    agg = p_ref[0] + p_ref[1]
    deg = deg_ref[0] + deg_ref[1]
    agg = agg / jnp.maximum(deg, 1.0)
    t = _lrelu(jnp.dot(agg, w1_ref[...],
                       preferred_element_type=jnp.float32) + b1_ref[...])
    h = _lrelu(jnp.dot(t, w2_ref[...],
                       preferred_element_type=jnp.float32) + b2_ref[...])
    o_ref[...] = jnp.dot(h, wro_ref[...],
                         preferred_element_type=jnp.float32) + bro_ref[...]


def _tc_layer_ro(partials, degp, w1, b1, w2, b2, wro, bro):
    o = wro.shape[1]
    return pl.pallas_call(
        _layer_ro_body,
        grid=(N // BLK,),
        in_specs=[
            pl.BlockSpec((NC, BLK, H), lambda i: (0, i, 0)),
            pl.BlockSpec((NC, BLK, 1), lambda i: (0, i, 0)),
            pl.BlockSpec((H, H), lambda i: (0, 0)),
            pl.BlockSpec((1, H), lambda i: (0, 0)),
            pl.BlockSpec((H, H), lambda i: (0, 0)),
            pl.BlockSpec((1, H), lambda i: (0, 0)),
            pl.BlockSpec((H, o), lambda i: (0, 0)),
            pl.BlockSpec((1, o), lambda i: (0, 0)),
        ],
        out_specs=pl.BlockSpec((BLK, o), lambda i: (i, 0)),
        out_shape=jax.ShapeDtypeStruct((N, o), jnp.float32),
    )(partials, degp, w1, b1.reshape(1, H), w2, b2.reshape(1, H),
      wro, bro.reshape(1, o))


def kernel(feats, edge_index, W_emb, b_emb, W1_0, b1_0, W2_0, b2_0,
           W1_1, b1_1, W2_1, b2_1, W_ro, b_ro):
    idx = jnp.stack([edge_index[0].reshape(NW, CHUNKS, C),
                     edge_index[1].reshape(NW, CHUNKS, C)], axis=2)
    zrows = jnp.zeros((ROWS_PER_TILE, H), jnp.float32)
    zdeg = jnp.zeros((N,), jnp.float32)

    aggp, degp = _sc_agg_deg(feats, idx, zrows, zdeg)
    degp3 = degp.reshape(NC, N, 1)
    h1 = _tc_layer1(aggp, degp3, W_emb, b_emb, W1_0, b1_0, W2_0, b2_0)
    aggp2 = _sc_agg(h1, idx, zrows, zdeg)
    return _tc_layer_ro(aggp2, degp3, W1_1, b1_1, W2_1, b2_1, W_ro, b_ro)


# speedup vs baseline: 1.0746x; 1.0203x over previous
import functools

import jax
import jax.numpy as jnp
from jax import lax
from jax.experimental import pallas as pl
from jax.experimental.pallas import tpu as pltpu
from jax.experimental.pallas import tpu_sc as plsc

N = 10000
E = N * 32
H = 128

NC = 2
NS = 16
NW = NC * NS
EPW = E // NW
C = 125
CHUNKS = EPW // C
NR = 3
ROWS_PER_TILE = 624
TAIL_ROWS = N - NS * ROWS_PER_TILE

_NEG_SLOPE = 0.2


def _lrelu(x):
    return jnp.where(x >= 0, x, _NEG_SLOPE * x)


def _make_sc_agg(compute_deg: bool):
    mesh = plsc.VectorSubcoreMesh(core_axis_name="c", subcore_axis_name="s")
    if compute_deg:
        out_type = [jax.ShapeDtypeStruct((NC, N, H), jnp.float32),
                    jax.ShapeDtypeStruct((NC, N), jnp.float32)]
    else:
        out_type = jax.ShapeDtypeStruct((NC, N, H), jnp.float32)
    scratch_types = (
        [pltpu.VMEM((2, C), jnp.int32) for _ in range(NR)]
        + [pltpu.VMEM((C, H), jnp.float32) for _ in range(NR)]
        + [
            pltpu.VMEM((128,), jnp.float32),
            pltpu.VMEM_SHARED((N, H), jnp.float32),
            pltpu.VMEM_SHARED((N,), jnp.float32),
        ]
        + [pltpu.SemaphoreType.DMA for _ in range(4 * NR)]
    )

    def body(h_hbm, idx_hbm, zrows_hbm, zdeg_hbm, *refs):
        if compute_deg:
            agg_out, deg_out = refs[0], refs[1]
            rest = refs[2:]
        else:
            agg_out = refs[0]
            deg_out = None
            rest = refs[1:]
        idxb = rest[:NR]
        rowsb = rest[NR:2 * NR]
        ones_v, agg_sh, deg_sh = rest[2 * NR:2 * NR + 3]
        sems = rest[2 * NR + 3:]
        gsem = sems[:NR]
        ssem = sems[NR:2 * NR]
        dsem = sems[2 * NR:3 * NR]
        isem = sems[3 * NR:]

        c = lax.axis_index("c")
        s = lax.axis_index("s")
        wid = c * NS + s

        for m in range(2):
            pltpu.async_copy(idx_hbm.at[wid, m], idxb[m], isem[m])

        pltpu.sync_copy(zrows_hbm,
                        agg_sh.at[pl.ds(s * ROWS_PER_TILE, ROWS_PER_TILE)])

        @pl.when(s == NS - 1)
        def _():
            pltpu.sync_copy(zrows_hbm.at[pl.ds(0, TAIL_ROWS)],
                            agg_sh.at[pl.ds(NS * ROWS_PER_TILE, TAIL_ROWS)])

        if compute_deg:
            @pl.when(s == 0)
            def _():
                pltpu.sync_copy(zdeg_hbm, deg_sh)
            one16 = jnp.ones((16,), jnp.float32)
            for j in range(8):
                ones_v[pl.ds(j * 16, 16)] = one16

        for m in range(2):
            pltpu.make_async_copy(idx_hbm.at[wid, m], idxb[m], isem[m]).wait()
            pltpu.async_copy(h_hbm.at[idxb[m].at[0]], rowsb[m], gsem[m])
        plsc.subcore_barrier()

        def emit_step(q, b, first, gather):
            bp = (b + NR - 1) % NR
            pltpu.make_async_copy(h_hbm.at[idxb[b].at[0]],
                                  rowsb[b], gsem[b]).wait()
            pltpu.async_copy(rowsb[b], agg_sh.at[idxb[b].at[1]],
                             ssem[b], add=True)
            if compute_deg:
                pltpu.async_copy(ones_v.at[pl.ds(0, C)],
                                 deg_sh.at[idxb[b].at[1]], dsem[b], add=True)
            if not first:
                pltpu.make_async_copy(rowsb[bp], agg_sh.at[idxb[bp].at[1]],
                                      ssem[bp]).wait()
                if compute_deg:
                    pltpu.make_async_copy(ones_v.at[pl.ds(0, C)],
                                          deg_sh.at[idxb[bp].at[1]],
                                          dsem[bp]).wait()

            def advance():
                pltpu.async_copy(idx_hbm.at[wid, q + 2], idxb[bp], isem[bp])
                pltpu.make_async_copy(idx_hbm.at[wid, q + 2],
                                      idxb[bp], isem[bp]).wait()
                pltpu.async_copy(h_hbm.at[idxb[bp].at[0]], rowsb[bp], gsem[bp])

            if gather == "dyn":
                pl.when(q + 2 < CHUNKS)(advance)
            elif gather:
                advance()

        for p in range(NR):
            emit_step(p, p, first=(p == 0), gather=True)

        def round_body(j, carry):
            q0 = j * NR
            for p in range(NR):
                emit_step(q0 + p, p, first=False, gather="dyn")
            return carry

        lax.fori_loop(1, CHUNKS // NR, round_body, 0)

        for q in range(NR * (CHUNKS // NR), CHUNKS):
            emit_step(q, q % NR, first=False, gather=False)
        b = (CHUNKS - 1) % NR
        pltpu.make_async_copy(rowsb[b], agg_sh.at[idxb[b].at[1]],
                              ssem[b]).wait()
        if compute_deg:
            pltpu.make_async_copy(ones_v.at[pl.ds(0, C)],
                                  deg_sh.at[idxb[b].at[1]], dsem[b]).wait()
        plsc.subcore_barrier()

        r0 = s * ROWS_PER_TILE
        pltpu.sync_copy(agg_sh.at[pl.ds(r0, ROWS_PER_TILE)],
                        agg_out.at[c, pl.ds(r0, ROWS_PER_TILE)])

        @pl.when(s == NS - 1)
        def _():
            rt = NS * ROWS_PER_TILE
            pltpu.sync_copy(agg_sh.at[pl.ds(rt, TAIL_ROWS)],
                            agg_out.at[c, pl.ds(rt, TAIL_ROWS)])

        if compute_deg:
            @pl.when(s == 0)
            def _():
                pltpu.sync_copy(deg_sh, deg_out.at[c])

    return functools.partial(pl.kernel, mesh=mesh, out_type=out_type,
                             scratch_types=scratch_types)(body)


_sc_agg_deg = _make_sc_agg(True)
_sc_agg = _make_sc_agg(False)


BLK = 2000


def _layer1_body(p_ref, deg_ref, we_ref, be_ref, w1_ref, b1_ref,
                 w2_ref, b2_ref, o_ref):
    deg = deg_ref[0] + deg_ref[1]
    aggf = (p_ref[0] + p_ref[1]) / jnp.maximum(deg, 1.0)
    agg = jnp.dot(aggf, we_ref[...],
                  preferred_element_type=jnp.float32) + be_ref[...]
    t = _lrelu(jnp.dot(agg, w1_ref[...],
                       preferred_element_type=jnp.float32) + b1_ref[...])
    o_ref[...] = _lrelu(jnp.dot(t, w2_ref[...],
                                preferred_element_type=jnp.float32) + b2_ref[...])


def _tc_layer1(partials, degp, we, be, w1, b1, w2, b2):
    d = we.shape[0]
    return pl.pallas_call(
        _layer1_body,
        grid=(N // BLK,),
        in_specs=[
            pl.BlockSpec((NC, BLK, d), lambda i: (0, i, 0)),
            pl.BlockSpec((NC, BLK, 1), lambda i: (0, i, 0)),
            pl.BlockSpec((d, H), lambda i: (0, 0)),
            pl.BlockSpec((1, H), lambda i: (0, 0)),
            pl.BlockSpec((H, H), lambda i: (0, 0)),
            pl.BlockSpec((1, H), lambda i: (0, 0)),
            pl.BlockSpec((H, H), lambda i: (0, 0)),
            pl.BlockSpec((1, H), lambda i: (0, 0)),
        ],
        out_specs=pl.BlockSpec((BLK, H), lambda i: (i, 0)),
        out_shape=jax.ShapeDtypeStruct((N, H), jnp.float32),
    )(partials, degp, we, be.reshape(1, H), w1, b1.reshape(1, H),
      w2, b2.reshape(1, H))


def _layer_ro_body(p_ref, deg_ref, w1_ref, b1_ref, w2_ref, b2_ref,
                   wro_ref, bro_ref, o_ref):
    agg = p_ref[0] + p_ref[1]
    deg = deg_ref[0] + deg_ref[1]
    agg = agg / jnp.maximum(deg, 1.0)
    t = _lrelu(jnp.dot(agg, w1_ref[...],
                       preferred_element_type=jnp.float32) + b1_ref[...])
    h = _lrelu(jnp.dot(t, w2_ref[...],
                       preferred_element_type=jnp.float32) + b2_ref[...])
    o_ref[...] = jnp.dot(h, wro_ref[...],
                         preferred_element_type=jnp.float32) + bro_ref[...]


def _tc_layer_ro(partials, degp, w1, b1, w2, b2, wro, bro):
    o = wro.shape[1]
    return pl.pallas_call(
        _layer_ro_body,
        grid=(N // BLK,),
        in_specs=[
            pl.BlockSpec((NC, BLK, H), lambda i: (0, i, 0)),
            pl.BlockSpec((NC, BLK, 1), lambda i: (0, i, 0)),
            pl.BlockSpec((H, H), lambda i: (0, 0)),
            pl.BlockSpec((1, H), lambda i: (0, 0)),
            pl.BlockSpec((H, H), lambda i: (0, 0)),
            pl.BlockSpec((1, H), lambda i: (0, 0)),
            pl.BlockSpec((H, o), lambda i: (0, 0)),
            pl.BlockSpec((1, o), lambda i: (0, 0)),
        ],
        out_specs=pl.BlockSpec((BLK, o), lambda i: (i, 0)),
        out_shape=jax.ShapeDtypeStruct((N, o), jnp.float32),
    )(partials, degp, w1, b1.reshape(1, H), w2, b2.reshape(1, H),
      wro, bro.reshape(1, o))


def kernel(feats, edge_index, W_emb, b_emb, W1_0, b1_0, W2_0, b2_0,
           W1_1, b1_1, W2_1, b2_1, W_ro, b_ro):
    idx = jnp.stack([edge_index[0].reshape(NW, CHUNKS, C),
                     edge_index[1].reshape(NW, CHUNKS, C)], axis=2)
    zrows = jnp.zeros((ROWS_PER_TILE, H), jnp.float32)
    zdeg = jnp.zeros((N,), jnp.float32)

    aggp, degp = _sc_agg_deg(feats, idx, zrows, zdeg)
    degp3 = degp.reshape(NC, N, 1)
    h1 = _tc_layer1(aggp, degp3, W_emb, b_emb, W1_0, b1_0, W2_0, b2_0)
    aggp2 = _sc_agg(h1, idx, zrows, zdeg)
    return _tc_layer_ro(aggp2, degp3, W1_1, b1_1, W2_1, b2_1, W_ro, b_ro)


# speedup vs baseline: 1.0822x; 1.0071x over previous
import functools

import jax
import jax.numpy as jnp
from jax import lax
from jax.experimental import pallas as pl
from jax.experimental.pallas import tpu as pltpu
from jax.experimental.pallas import tpu_sc as plsc

N = 10000
E = N * 32
H = 128

NC = 2
NS = 16
NW = NC * NS
EPW = E // NW
C = 125
CHUNKS = EPW // C
NR = 3
ROWS_PER_TILE = 624
TAIL_ROWS = N - NS * ROWS_PER_TILE

_NEG_SLOPE = 0.2


def _lrelu(x):
    return jnp.where(x >= 0, x, _NEG_SLOPE * x)


def _make_sc_agg(compute_deg: bool):
    mesh = plsc.VectorSubcoreMesh(core_axis_name="c", subcore_axis_name="s")
    if compute_deg:
        out_type = [jax.ShapeDtypeStruct((NC, N, H), jnp.float32),
                    jax.ShapeDtypeStruct((NC, N), jnp.float32)]
    else:
        out_type = jax.ShapeDtypeStruct((NC, N, H), jnp.float32)
    scratch_types = (
        [pltpu.VMEM((2, C), jnp.int32) for _ in range(NR)]
        + [pltpu.VMEM((C, H), jnp.float32) for _ in range(NR)]
        + [
            pltpu.VMEM((128,), jnp.float32),
            pltpu.VMEM_SHARED((N, H), jnp.float32),
            pltpu.VMEM_SHARED((N,), jnp.float32),
        ]
        + [pltpu.SemaphoreType.DMA for _ in range(4 * NR)]
    )

    def body(h_hbm, idx_hbm, zrows_hbm, zdeg_hbm, *refs):
        if compute_deg:
            agg_out, deg_out = refs[0], refs[1]
            rest = refs[2:]
        else:
            agg_out = refs[0]
            deg_out = None
            rest = refs[1:]
        idxb = rest[:NR]
        rowsb = rest[NR:2 * NR]
        ones_v, agg_sh, deg_sh = rest[2 * NR:2 * NR + 3]
        sems = rest[2 * NR + 3:]
        gsem = sems[:NR]
        ssem = sems[NR:2 * NR]
        dsem = sems[2 * NR:3 * NR]
        isem = sems[3 * NR:]

        c = lax.axis_index("c")
        s = lax.axis_index("s")
        wid = c * NS + s

        for m in range(2):
            pltpu.async_copy(idx_hbm.at[wid, m], idxb[m], isem[m])

        pltpu.sync_copy(zrows_hbm,
                        agg_sh.at[pl.ds(s * ROWS_PER_TILE, ROWS_PER_TILE)])

        @pl.when(s == NS - 1)
        def _():
            pltpu.sync_copy(zrows_hbm.at[pl.ds(0, TAIL_ROWS)],
                            agg_sh.at[pl.ds(NS * ROWS_PER_TILE, TAIL_ROWS)])

        if compute_deg:
            @pl.when(s == 0)
            def _():
                pltpu.sync_copy(zdeg_hbm, deg_sh)
            one16 = jnp.ones((16,), jnp.float32)
            for j in range(8):
                ones_v[pl.ds(j * 16, 16)] = one16

        for m in range(2):
            pltpu.make_async_copy(idx_hbm.at[wid, m], idxb[m], isem[m]).wait()
            pltpu.async_copy(h_hbm.at[idxb[m].at[0]], rowsb[m], gsem[m])
        plsc.subcore_barrier()

        def emit_step(q, b, first, gather):
            bp = (b + NR - 1) % NR
            pltpu.make_async_copy(h_hbm.at[idxb[b].at[0]],
                                  rowsb[b], gsem[b]).wait()
            pltpu.async_copy(rowsb[b], agg_sh.at[idxb[b].at[1]],
                             ssem[b], add=True)
            if compute_deg:
                pltpu.async_copy(ones_v.at[pl.ds(0, C)],
                                 deg_sh.at[idxb[b].at[1]], dsem[b], add=True)
            if not first:
                pltpu.make_async_copy(rowsb[bp], agg_sh.at[idxb[bp].at[1]],
                                      ssem[bp]).wait()
                if compute_deg:
                    pltpu.make_async_copy(ones_v.at[pl.ds(0, C)],
                                          deg_sh.at[idxb[bp].at[1]],
                                          dsem[bp]).wait()

            def advance():
                pltpu.async_copy(idx_hbm.at[wid, q + 2], idxb[bp], isem[bp])
                pltpu.make_async_copy(idx_hbm.at[wid, q + 2],
                                      idxb[bp], isem[bp]).wait()
                pltpu.async_copy(h_hbm.at[idxb[bp].at[0]], rowsb[bp], gsem[bp])

            if gather == "dyn":
                pl.when(q + 2 < CHUNKS)(advance)
            elif gather:
                advance()

        for p in range(NR):
            emit_step(p, p, first=(p == 0), gather=True)

        def round_body(j, carry):
            q0 = j * NR
            for p in range(NR):
                emit_step(q0 + p, p, first=False, gather="dyn")
            return carry

        lax.fori_loop(1, CHUNKS // NR, round_body, 0)

        for q in range(NR * (CHUNKS // NR), CHUNKS):
            emit_step(q, q % NR, first=False, gather=False)
        b = (CHUNKS - 1) % NR
        pltpu.make_async_copy(rowsb[b], agg_sh.at[idxb[b].at[1]],
                              ssem[b]).wait()
        if compute_deg:
            pltpu.make_async_copy(ones_v.at[pl.ds(0, C)],
                                  deg_sh.at[idxb[b].at[1]], dsem[b]).wait()
        plsc.subcore_barrier()

        r0 = s * ROWS_PER_TILE
        pltpu.sync_copy(agg_sh.at[pl.ds(r0, ROWS_PER_TILE)],
                        agg_out.at[c, pl.ds(r0, ROWS_PER_TILE)])

        @pl.when(s == NS - 1)
        def _():
            rt = NS * ROWS_PER_TILE
            pltpu.sync_copy(agg_sh.at[pl.ds(rt, TAIL_ROWS)],
                            agg_out.at[c, pl.ds(rt, TAIL_ROWS)])

        if compute_deg:
            @pl.when(s == 0)
            def _():
                pltpu.sync_copy(deg_sh, deg_out.at[c])

    return functools.partial(pl.kernel, mesh=mesh, out_type=out_type,
                             scratch_types=scratch_types)(body)


_sc_agg_deg = _make_sc_agg(True)
_sc_agg = _make_sc_agg(False)


BLK = 5000


def _layer1_body(p_ref, deg_ref, we_ref, be_ref, w1_ref, b1_ref,
                 w2_ref, b2_ref, o_ref):
    deg = deg_ref[0] + deg_ref[1]
    aggf = (p_ref[0] + p_ref[1]) / jnp.maximum(deg, 1.0)
    agg = jnp.dot(aggf, we_ref[...],
                  preferred_element_type=jnp.float32) + be_ref[...]
    t = _lrelu(jnp.dot(agg, w1_ref[...],
                       preferred_element_type=jnp.float32) + b1_ref[...])
    o_ref[...] = _lrelu(jnp.dot(t, w2_ref[...],
                                preferred_element_type=jnp.float32) + b2_ref[...])


def _tc_layer1(partials, degp, we, be, w1, b1, w2, b2):
    d = we.shape[0]
    return pl.pallas_call(
        _layer1_body,
        grid=(N // BLK,),
        in_specs=[
            pl.BlockSpec((NC, BLK, d), lambda i: (0, i, 0)),
            pl.BlockSpec((NC, BLK, 1), lambda i: (0, i, 0)),
            pl.BlockSpec((d, H), lambda i: (0, 0)),
            pl.BlockSpec((1, H), lambda i: (0, 0)),
            pl.BlockSpec((H, H), lambda i: (0, 0)),
            pl.BlockSpec((1, H), lambda i: (0, 0)),
            pl.BlockSpec((H, H), lambda i: (0, 0)),
            pl.BlockSpec((1, H), lambda i: (0, 0)),
        ],
        out_specs=pl.BlockSpec((BLK, H), lambda i: (i, 0)),
        out_shape=jax.ShapeDtypeStruct((N, H), jnp.float32),
    )(partials, degp, we, be.reshape(1, H), w1, b1.reshape(1, H),
      w2, b2.reshape(1, H))


def _layer_ro_body(p_ref, deg_ref, w1_ref, b1_ref, w2_ref, b2_ref,
                   wro_ref, bro_ref, o_ref):
    agg = p_ref[0] + p_ref[1]
    deg = deg_ref[0] + deg_ref[1]
    agg = agg / jnp.maximum(deg, 1.0)
    t = _lrelu(jnp.dot(agg, w1_ref[...],
                       preferred_element_type=jnp.float32) + b1_ref[...])
    h = _lrelu(jnp.dot(t, w2_ref[...],
                       preferred_element_type=jnp.float32) + b2_ref[...])
    o_ref[...] = jnp.dot(h, wro_ref[...],
                         preferred_element_type=jnp.float32) + bro_ref[...]


def _tc_layer_ro(partials, degp, w1, b1, w2, b2, wro, bro):
    o = wro.shape[1]
    return pl.pallas_call(
        _layer_ro_body,
        grid=(N // BLK,),
        in_specs=[
            pl.BlockSpec((NC, BLK, H), lambda i: (0, i, 0)),
            pl.BlockSpec((NC, BLK, 1), lambda i: (0, i, 0)),
            pl.BlockSpec((H, H), lambda i: (0, 0)),
            pl.BlockSpec((1, H), lambda i: (0, 0)),
            pl.BlockSpec((H, H), lambda i: (0, 0)),
            pl.BlockSpec((1, H), lambda i: (0, 0)),
            pl.BlockSpec((H, o), lambda i: (0, 0)),
            pl.BlockSpec((1, o), lambda i: (0, 0)),
        ],
        out_specs=pl.BlockSpec((BLK, o), lambda i: (i, 0)),
        out_shape=jax.ShapeDtypeStruct((N, o), jnp.float32),
    )(partials, degp, w1, b1.reshape(1, H), w2, b2.reshape(1, H),
      wro, bro.reshape(1, o))


def kernel(feats, edge_index, W_emb, b_emb, W1_0, b1_0, W2_0, b2_0,
           W1_1, b1_1, W2_1, b2_1, W_ro, b_ro):
    idx = jnp.stack([edge_index[0].reshape(NW, CHUNKS, C),
                     edge_index[1].reshape(NW, CHUNKS, C)], axis=2)
    zrows = jnp.zeros((ROWS_PER_TILE, H), jnp.float32)
    zdeg = jnp.zeros((N,), jnp.float32)

    aggp, degp = _sc_agg_deg(feats, idx, zrows, zdeg)
    degp3 = degp.reshape(NC, N, 1)
    h1 = _tc_layer1(aggp, degp3, W_emb, b_emb, W1_0, b1_0, W2_0, b2_0)
    aggp2 = _sc_agg(h1, idx, zrows, zdeg)
    return _tc_layer_ro(aggp2, degp3, W1_1, b1_1, W2_1, b2_1, W_ro, b_ro)


# speedup vs baseline: 1.1518x; 1.0644x over previous
import functools

import jax
import jax.numpy as jnp
from jax import lax
from jax.experimental import pallas as pl
from jax.experimental.pallas import tpu as pltpu
from jax.experimental.pallas import tpu_sc as plsc

N = 10000
E = N * 32
H = 128

NC = 2
NS = 16
NW = NC * NS
EPW = E // NW
C = 125
CHUNKS = EPW // C
NR = 3
ROWS_PER_TILE = 624
TAIL_ROWS = N - NS * ROWS_PER_TILE

_NEG_SLOPE = 0.2


def _lrelu(x):
    return jnp.where(x >= 0, x, _NEG_SLOPE * x)


def _make_sc_agg(compute_deg: bool):
    mesh = plsc.VectorSubcoreMesh(core_axis_name="c", subcore_axis_name="s")
    if compute_deg:
        out_type = [jax.ShapeDtypeStruct((NC, N, H), jnp.float32),
                    jax.ShapeDtypeStruct((NC, N), jnp.float32)]
    else:
        out_type = jax.ShapeDtypeStruct((NC, N, H), jnp.float32)
    scratch_types = (
        [pltpu.VMEM((2, C), jnp.int32) for _ in range(NR)]
        + [pltpu.VMEM((C, H), jnp.float32) for _ in range(NR)]
        + [
            pltpu.VMEM((128,), jnp.float32),
            pltpu.VMEM_SHARED((N, H), jnp.float32),
            pltpu.VMEM_SHARED((N,), jnp.float32),
        ]
        + [pltpu.SemaphoreType.DMA for _ in range(4 * NR)]
    )

    def body(h_hbm, idx_hbm, zrows_hbm, zdeg_hbm, *refs):
        if compute_deg:
            agg_out, deg_out = refs[0], refs[1]
            rest = refs[2:]
        else:
            agg_out = refs[0]
            deg_out = None
            rest = refs[1:]
        idxb = rest[:NR]
        rowsb = rest[NR:2 * NR]
        ones_v, agg_sh, deg_sh = rest[2 * NR:2 * NR + 3]
        sems = rest[2 * NR + 3:]
        gsem = sems[:NR]
        ssem = sems[NR:2 * NR]
        dsem = sems[2 * NR:3 * NR]
        isem = sems[3 * NR:]

        c = lax.axis_index("c")
        s = lax.axis_index("s")
        wid = c * NS + s

        for m in range(2):
            pltpu.async_copy(idx_hbm.at[0, wid, m], idxb[m].at[0], isem[m])
            pltpu.async_copy(idx_hbm.at[1, wid, m], idxb[m].at[1], isem[m])

        pltpu.sync_copy(zrows_hbm,
                        agg_sh.at[pl.ds(s * ROWS_PER_TILE, ROWS_PER_TILE)])

        @pl.when(s == NS - 1)
        def _():
            pltpu.sync_copy(zrows_hbm.at[pl.ds(0, TAIL_ROWS)],
                            agg_sh.at[pl.ds(NS * ROWS_PER_TILE, TAIL_ROWS)])

        if compute_deg:
            @pl.when(s == 0)
            def _():
                pltpu.sync_copy(zdeg_hbm, deg_sh)
            one16 = jnp.ones((16,), jnp.float32)
            for j in range(8):
                ones_v[pl.ds(j * 16, 16)] = one16

        for m in range(2):
            pltpu.make_async_copy(idx_hbm.at[0, wid, m],
                                  idxb[m].at[0], isem[m]).wait()
            pltpu.make_async_copy(idx_hbm.at[1, wid, m],
                                  idxb[m].at[1], isem[m]).wait()
            pltpu.async_copy(h_hbm.at[idxb[m].at[0]], rowsb[m], gsem[m])
        plsc.subcore_barrier()

        def emit_step(q, b, first, gather):
            bp = (b + NR - 1) % NR
            pltpu.make_async_copy(h_hbm.at[idxb[b].at[0]],
                                  rowsb[b], gsem[b]).wait()
            pltpu.async_copy(rowsb[b], agg_sh.at[idxb[b].at[1]],
                             ssem[b], add=True)
            if compute_deg:
                pltpu.async_copy(ones_v.at[pl.ds(0, C)],
                                 deg_sh.at[idxb[b].at[1]], dsem[b], add=True)
            if not first:
                pltpu.make_async_copy(rowsb[bp], agg_sh.at[idxb[bp].at[1]],
                                      ssem[bp]).wait()
                if compute_deg:
                    pltpu.make_async_copy(ones_v.at[pl.ds(0, C)],
                                          deg_sh.at[idxb[bp].at[1]],
                                          dsem[bp]).wait()

            def advance():
                pltpu.async_copy(idx_hbm.at[0, wid, q + 2],
                                 idxb[bp].at[0], isem[bp])
                pltpu.async_copy(idx_hbm.at[1, wid, q + 2],
                                 idxb[bp].at[1], isem[bp])
                pltpu.make_async_copy(idx_hbm.at[0, wid, q + 2],
                                      idxb[bp].at[0], isem[bp]).wait()
                pltpu.make_async_copy(idx_hbm.at[1, wid, q + 2],
                                      idxb[bp].at[1], isem[bp]).wait()
                pltpu.async_copy(h_hbm.at[idxb[bp].at[0]], rowsb[bp], gsem[bp])

            if gather == "dyn":
                pl.when(q + 2 < CHUNKS)(advance)
            elif gather:
                advance()

        for p in range(NR):
            emit_step(p, p, first=(p == 0), gather=True)

        def round_body(j, carry):
            q0 = j * NR
            for p in range(NR):
                emit_step(q0 + p, p, first=False, gather="dyn")
            return carry

        lax.fori_loop(1, CHUNKS // NR, round_body, 0)

        for q in range(NR * (CHUNKS // NR), CHUNKS):
            emit_step(q, q % NR, first=False, gather=False)
        b = (CHUNKS - 1) % NR
        pltpu.make_async_copy(rowsb[b], agg_sh.at[idxb[b].at[1]],
                              ssem[b]).wait()
        if compute_deg:
            pltpu.make_async_copy(ones_v.at[pl.ds(0, C)],
                                  deg_sh.at[idxb[b].at[1]], dsem[b]).wait()
        plsc.subcore_barrier()

        r0 = s * ROWS_PER_TILE
        pltpu.sync_copy(agg_sh.at[pl.ds(r0, ROWS_PER_TILE)],
                        agg_out.at[c, pl.ds(r0, ROWS_PER_TILE)])

        @pl.when(s == NS - 1)
        def _():
            rt = NS * ROWS_PER_TILE
            pltpu.sync_copy(agg_sh.at[pl.ds(rt, TAIL_ROWS)],
                            agg_out.at[c, pl.ds(rt, TAIL_ROWS)])

        if compute_deg:
            @pl.when(s == 0)
            def _():
                pltpu.sync_copy(deg_sh, deg_out.at[c])

    return functools.partial(pl.kernel, mesh=mesh, out_type=out_type,
                             scratch_types=scratch_types)(body)


_sc_agg_deg = _make_sc_agg(True)
_sc_agg = _make_sc_agg(False)


BLK = 5000


def _layer1_body(p_ref, deg_ref, we_ref, be_ref, w1_ref, b1_ref,
                 w2_ref, b2_ref, o_ref):
    deg = deg_ref[0] + deg_ref[1]
    aggf = (p_ref[0] + p_ref[1]) / jnp.maximum(deg, 1.0)
    agg = jnp.dot(aggf, we_ref[...],
                  preferred_element_type=jnp.float32) + be_ref[...]
    t = _lrelu(jnp.dot(agg, w1_ref[...],
                       preferred_element_type=jnp.float32) + b1_ref[...])
    o_ref[...] = _lrelu(jnp.dot(t, w2_ref[...],
                                preferred_element_type=jnp.float32) + b2_ref[...])


def _tc_layer1(partials, degp, we, be, w1, b1, w2, b2):
    d = we.shape[0]
    return pl.pallas_call(
        _layer1_body,
        grid=(N // BLK,),
        in_specs=[
            pl.BlockSpec((NC, BLK, d), lambda i: (0, i, 0)),
            pl.BlockSpec((NC, BLK, 1), lambda i: (0, i, 0)),
            pl.BlockSpec((d, H), lambda i: (0, 0)),
            pl.BlockSpec((1, H), lambda i: (0, 0)),
            pl.BlockSpec((H, H), lambda i: (0, 0)),
            pl.BlockSpec((1, H), lambda i: (0, 0)),
            pl.BlockSpec((H, H), lambda i: (0, 0)),
            pl.BlockSpec((1, H), lambda i: (0, 0)),
        ],
        out_specs=pl.BlockSpec((BLK, H), lambda i: (i, 0)),
        out_shape=jax.ShapeDtypeStruct((N, H), jnp.float32),
    )(partials, degp, we, be.reshape(1, H), w1, b1.reshape(1, H),
      w2, b2.reshape(1, H))


def _layer_ro_body(p_ref, deg_ref, w1_ref, b1_ref, w2_ref, b2_ref,
                   wro_ref, bro_ref, o_ref):
    agg = p_ref[0] + p_ref[1]
    deg = deg_ref[0] + deg_ref[1]
    agg = agg / jnp.maximum(deg, 1.0)
    t = _lrelu(jnp.dot(agg, w1_ref[...],
                       preferred_element_type=jnp.float32) + b1_ref[...])
    h = _lrelu(jnp.dot(t, w2_ref[...],
                       preferred_element_type=jnp.float32) + b2_ref[...])
    o_ref[...] = jnp.dot(h, wro_ref[...],
                         preferred_element_type=jnp.float32) + bro_ref[...]


def _tc_layer_ro(partials, degp, w1, b1, w2, b2, wro, bro):
    o = wro.shape[1]
    return pl.pallas_call(
        _layer_ro_body,
        grid=(N // BLK,),
        in_specs=[
            pl.BlockSpec((NC, BLK, H), lambda i: (0, i, 0)),
            pl.BlockSpec((NC, BLK, 1), lambda i: (0, i, 0)),
            pl.BlockSpec((H, H), lambda i: (0, 0)),
            pl.BlockSpec((1, H), lambda i: (0, 0)),
            pl.BlockSpec((H, H), lambda i: (0, 0)),
            pl.BlockSpec((1, H), lambda i: (0, 0)),
            pl.BlockSpec((H, o), lambda i: (0, 0)),
            pl.BlockSpec((1, o), lambda i: (0, 0)),
        ],
        out_specs=pl.BlockSpec((BLK, o), lambda i: (i, 0)),
        out_shape=jax.ShapeDtypeStruct((N, o), jnp.float32),
    )(partials, degp, w1, b1.reshape(1, H), w2, b2.reshape(1, H),
      wro, bro.reshape(1, o))


def kernel(feats, edge_index, W_emb, b_emb, W1_0, b1_0, W2_0, b2_0,
           W1_1, b1_1, W2_1, b2_1, W_ro, b_ro):
    idx = edge_index.reshape(2, NW, CHUNKS, C)
    zrows = jnp.zeros((ROWS_PER_TILE, H), jnp.float32)
    zdeg = jnp.zeros((N,), jnp.float32)

    aggp, degp = _sc_agg_deg(feats, idx, zrows, zdeg)
    degp3 = degp.reshape(NC, N, 1)
    h1 = _tc_layer1(aggp, degp3, W_emb, b_emb, W1_0, b1_0, W2_0, b2_0)
    aggp2 = _sc_agg(h1, idx, zrows, zdeg)
    return _tc_layer_ro(aggp2, degp3, W1_1, b1_1, W2_1, b2_1, W_ro, b_ro)
